# baseline (speedup 1.0000x reference)
import functools

import jax
import jax.numpy as jnp
from jax import lax
from jax.experimental import pallas as pl
from jax.experimental.pallas import tpu as pltpu
from jax.experimental.pallas import tpu_sc as plsc

_B, _L, _V, _D = 1024, 200, 100000, 128
_NC, _NS = 2, 16
_NW = _NC * _NS
_N = _B * _L
_PER_W = _N // _NW
_CHUNK = 320
_NCHUNKS = _PER_W // _CHUNK


def _sort_permute_body(inp_ref, len_col_ref, len_row_ref, tgt_col_ref,
                       tok_ref, aux_ref):
    lens_col = jnp.maximum(len_col_ref[...], 1)
    lens_row = jnp.maximum(len_row_ref[...], 1)
    row_i = lax.broadcasted_iota(jnp.int32, (_B, 1), 0)
    col_i = lax.broadcasted_iota(jnp.int32, (1, _B), 1)
    key_col = (_L - lens_col) * _B + row_i
    key_row = (_L - lens_row) * _B + col_i
    m = (key_row < key_col).astype(jnp.int32)
    rank_col = jnp.sum(m, axis=1, keepdims=True)
    pt = (rank_col == col_i).astype(jnp.float32)
    p_i = lax.broadcasted_iota(jnp.int32, (_B, _L), 1)
    toks = jnp.where(p_i < lens_col, inp_ref[...], 0).astype(jnp.float32)
    dn = (((0,), (0,)), ((), ()))
    tok_sorted = lax.dot_general(pt, toks, dn,
                                 precision=lax.Precision.HIGHEST,
                                 preferred_element_type=jnp.float32)
    tok_ref[...] = (tok_sorted + 0.5).astype(jnp.int32)
    lane_i = lax.broadcasted_iota(jnp.int32, (_B, _D), 1)
    aux = jnp.where(lane_i == 0, lens_col.astype(jnp.float32),
                    jnp.where(lane_i == 1, tgt_col_ref[...], 0.0))
    aux_ref[...] = lax.dot_general(pt, aux, dn,
                                   precision=lax.Precision.HIGHEST,
                                   preferred_element_type=jnp.float32)


def _gather_body(idx_hbm, table_hbm, out_hbm,
                 idx_v, buf0, buf1, gs0, gs1, ws0, ws1):
    wid = lax.axis_index("s") * _NC + lax.axis_index("c")
    base = wid * _PER_W
    pltpu.sync_copy(idx_hbm.at[pl.ds(base, _PER_W)], idx_v)
    bufs, gsems, wsems = (buf0, buf1), (gs0, gs1), (ws0, ws1)
    gops = [pltpu.async_copy(table_hbm.at[idx_v.at[pl.ds(0, _CHUNK)]],
                             buf0, gs0), None]
    wops = [None, None]
    for c in range(_NCHUNKS):
        b = c % 2
        nb = 1 - b
        if c + 1 < _NCHUNKS:
            if c >= 1:
                wops[nb].wait()
            gops[nb] = pltpu.async_copy(
                table_hbm.at[idx_v.at[pl.ds((c + 1) * _CHUNK, _CHUNK)]],
                bufs[nb], gsems[nb])
        gops[b].wait()
        wops[b] = pltpu.async_copy(
            bufs[b], out_hbm.at[pl.ds(base + c * _CHUNK, _CHUNK)], wsems[b])
    wops[(_NCHUNKS - 1) % 2].wait()
    wops[(_NCHUNKS - 2) % 2].wait()


@functools.cache
def _gather_call():
    return functools.partial(
        pl.kernel,
        mesh=plsc.VectorSubcoreMesh(core_axis_name="c", subcore_axis_name="s"),
        out_type=jax.ShapeDtypeStruct((_N, _D), jnp.float32),
        scratch_types=[
            pltpu.VMEM((_PER_W,), jnp.int32),
            pltpu.VMEM((_CHUNK, _D), jnp.float32),
            pltpu.VMEM((_CHUNK, _D), jnp.float32),
            pltpu.SemaphoreType.DMA,
            pltpu.SemaphoreType.DMA,
            pltpu.SemaphoreType.DMA,
            pltpu.SemaphoreType.DMA,
        ],
    )(_gather_body)


def kernel(input_batch, seq_lengths, targets_batch, table):
    inp = input_batch.astype(jnp.int32)
    sl = seq_lengths.astype(jnp.int32)
    tgt = targets_batch.astype(jnp.float32)
    tbl = table.astype(jnp.float32)
    tok_sorted, aux = pl.pallas_call(
        _sort_permute_body,
        out_shape=[jax.ShapeDtypeStruct((_B, _L), jnp.int32),
                   jax.ShapeDtypeStruct((_B, _D), jnp.float32)],
    )(inp, sl.reshape(_B, 1), sl.reshape(1, _B), tgt.reshape(_B, 1))
    flat_idx = tok_sorted.reshape(_N)
    out = _gather_call()(flat_idx, tbl)
    return out.reshape(_B, _L, _D), aux[:, 0], aux[:, 1]
